# Initial kernel scaffold; baseline (speedup 1.0000x reference)
import jax
import jax.numpy as jnp
from jax.experimental import pallas as pl


def kernel(q_len, k_len, bias_table):
    raise NotImplementedError("write your pallas kernel here")


# speedup vs baseline: 48.1469x; 48.1469x over previous
import functools

import jax
import jax.numpy as jnp
from jax import lax
from jax.experimental import pallas as pl
from jax.experimental.pallas import tpu as pltpu
from jax.experimental.pallas import tpu_sc as plsc

NUM_WORKERS = 32
K_INFLIGHT = 32
NSHIFT = 8


def _expand_kernel(n, nh, padded_row_words, rows_per_worker):
    rows_total = nh * n
    mesh = plsc.VectorSubcoreMesh(core_axis_name="c", subcore_axis_name="s")

    @functools.partial(
        pl.kernel,
        out_type=jax.ShapeDtypeStruct((rows_total * n,), jnp.float32),
        mesh=mesh,
        scratch_types=[
            pltpu.VMEM((NSHIFT * padded_row_words,), jnp.float32),
            pltpu.SemaphoreType.DMA,
        ],
    )
    def expand(padded_hbm, out_hbm, buf, sem):
        wid = lax.axis_index("s") * 2 + lax.axis_index("c")
        r0 = wid * rows_per_worker
        r1 = r0 + rows_per_worker
        for t in range(2):
            h = jnp.minimum(r0 // n + t, nh - 1)
            lo = jnp.maximum(r0, h * n)
            hi = jnp.minimum(r1, (h + 1) * n)

            @pl.when((r0 // n + t < nh) & (lo < hi))
            def _per_head(h=h, lo=lo, hi=hi):
                pltpu.sync_copy(padded_hbm.at[h], buf)

                def row_body(g, carry):
                    i = g - h * n
                    s = (n - 1) - i
                    r = (8 - (s & 7)) & 7
                    off = pl.multiple_of(r * (padded_row_words + 1) + s, 8)
                    dst = pl.multiple_of(g * n, n)
                    pltpu.async_copy(buf.at[pl.ds(off, n)],
                                     out_hbm.at[pl.ds(dst, n)], sem)

                    @pl.when(g >= lo + K_INFLIGHT)
                    def _drain_one():
                        pltpu.make_async_copy(
                            padded_hbm.at[h, pl.ds(0, n)],
                            buf.at[pl.ds(0, n)], sem).wait()

                    return carry

                lax.fori_loop(lo, hi, row_body, 0)

                def drain_body(_, carry):
                    pltpu.make_async_copy(
                        padded_hbm.at[h, pl.ds(0, n)],
                        buf.at[pl.ds(0, n)], sem).wait()
                    return carry

                lax.fori_loop(0, jnp.minimum(K_INFLIGHT, hi - lo), drain_body, 0)

    return expand


def kernel(q_len, k_len, bias_table):
    t_rows, nh = bias_table.shape
    n = (t_rows + 1) // 2
    assert (nh * n) % NUM_WORKERS == 0
    rows_per_worker = nh * n // NUM_WORKERS

    base = jnp.asarray(k_len, jnp.int32) - n + jnp.asarray(q_len, jnp.int32) - 1
    u = jnp.arange(2 * n - 1, dtype=jnp.int32)
    ext_idx = jnp.clip(u - (n - 1) + base, 0, t_rows - 1)
    ext_t = bias_table[ext_idx].T

    ext_len = 2 * n - 1
    row_words = ext_len + NSHIFT
    row_words = ((row_words + 7) // 8) * 8
    shifted = jnp.stack(
        [jnp.pad(ext_t, ((0, 0), (r, row_words - ext_len - r)))
         for r in range(NSHIFT)], axis=1)
    padded = shifted.reshape(nh, NSHIFT * row_words)

    expand = _expand_kernel(n, nh, row_words, rows_per_worker)
    out = expand(padded)
    return out.reshape(1, nh, n, n)
